# Initial kernel scaffold; baseline (speedup 1.0000x reference)
import jax
import jax.numpy as jnp
from jax.experimental import pallas as pl


def kernel(x, edge_index, edge_weight, W1, b1, W2, b2, W3, b3, g1, be1, g2, be2, Wh1, bh1, Wh2, bh2):
    raise NotImplementedError("write your pallas kernel here")


# speedup vs baseline: 11.7907x; 11.7907x over previous
import functools

import jax
import jax.numpy as jnp
from jax import lax
from jax.experimental import pallas as pl
from jax.experimental.pallas import tpu as pltpu
from jax.experimental.pallas import tpu_sc as plsc

N = 10000
E = 320000
EPS = 1e-5

NW = 32
B = 128
K = 80
E_PAD = NW * K * B
N_PAD = 10240
ROWS_PER_SUB = N_PAD // 16


def _sc_mesh():
    return plsc.VectorSubcoreMesh(core_axis_name="c", subcore_axis_name="s")


@functools.lru_cache(maxsize=None)
def _make_sc_deg():
    @functools.partial(
        pl.kernel,
        out_type=jax.ShapeDtypeStruct((2, N_PAD, 16), jnp.float32),
        mesh=_sc_mesh(),
        compiler_params=pltpu.CompilerParams(use_tc_tiling_on_sc=False),
        scratch_types=[
            pltpu.VMEM((K, B), jnp.int32),
            pltpu.VMEM((K, B), jnp.float32),
            pltpu.VMEM((B, 16), jnp.float32),
            pltpu.VMEM_SHARED((N_PAD, 16), jnp.float32),
        ],
    )
    def sc_deg(dst_hbm, ew_hbm, zeros_hbm, out_hbm, idx_d, ew_v, rows, acc):
        c = lax.axis_index("c")
        s = lax.axis_index("s")
        wid = s * 2 + c
        pltpu.sync_copy(dst_hbm.at[pl.ds(wid * K, K)], idx_d)
        pltpu.sync_copy(ew_hbm.at[pl.ds(wid * K, K)], ew_v)
        pltpu.sync_copy(zeros_hbm.at[pl.ds(s * ROWS_PER_SUB, ROWS_PER_SUB)],
                        acc.at[pl.ds(s * ROWS_PER_SUB, ROWS_PER_SUB)])
        plsc.subcore_barrier()

        def body(j, carry):
            for g in range(B // 16):
                ew16 = ew_v[j, pl.ds(g * 16, 16)]
                for l in range(16):
                    rows[g * 16 + l, :] = jnp.full((16,), ew16[l], jnp.float32)
            pltpu.sync_copy(rows, acc.at[idx_d.at[j]], add=True)
            return carry

        lax.fori_loop(0, K, body, 0)
        plsc.subcore_barrier()
        pltpu.sync_copy(acc.at[pl.ds(s * ROWS_PER_SUB, ROWS_PER_SUB)],
                        out_hbm.at[c, pl.ds(s * ROWS_PER_SUB, ROWS_PER_SUB)])

    return sc_deg


@functools.lru_cache(maxsize=None)
def _make_sc_layer(D):
    @functools.partial(
        pl.kernel,
        out_type=jax.ShapeDtypeStruct((2, N_PAD, D), jnp.float32),
        mesh=_sc_mesh(),
        compiler_params=pltpu.CompilerParams(use_tc_tiling_on_sc=False),
        scratch_types=[
            pltpu.VMEM((K, B), jnp.int32),
            pltpu.VMEM((K, B), jnp.int32),
            pltpu.VMEM((K, B), jnp.float32),
            pltpu.VMEM((B, D), jnp.float32),
            pltpu.VMEM_SHARED((N_PAD, D), jnp.float32),
            pltpu.SemaphoreType.DMA,
        ],
    )
    def sc_layer(src_hbm, dst_hbm, ew_hbm, hs_hbm, zeros_hbm, out_hbm,
                 idx_s, idx_d, ew_v, rows, acc, sem):
        c = lax.axis_index("c")
        s = lax.axis_index("s")
        wid = s * 2 + c
        pltpu.sync_copy(src_hbm.at[pl.ds(wid * K, K)], idx_s)
        pltpu.sync_copy(dst_hbm.at[pl.ds(wid * K, K)], idx_d)
        pltpu.sync_copy(ew_hbm.at[pl.ds(wid * K, K)], ew_v)
        pltpu.sync_copy(zeros_hbm.at[pl.ds(s * ROWS_PER_SUB, ROWS_PER_SUB)],
                        acc.at[pl.ds(s * ROWS_PER_SUB, ROWS_PER_SUB)])
        plsc.subcore_barrier()

        def body(j, carry):
            pltpu.async_copy(hs_hbm.at[idx_s.at[j]], rows, sem).wait()
            for g in range(B // 16):
                ew16 = ew_v[j, pl.ds(g * 16, 16)]
                for l in range(16):
                    e = g * 16 + l
                    w = jnp.full((16,), ew16[l], jnp.float32)
                    for d0 in range(0, D, 16):
                        rows[e, pl.ds(d0, 16)] = rows[e, pl.ds(d0, 16)] * w
            pltpu.sync_copy(rows, acc.at[idx_d.at[j]], add=True)
            return carry

        lax.fori_loop(0, K, body, 0)
        plsc.subcore_barrier()
        pltpu.sync_copy(acc.at[pl.ds(s * ROWS_PER_SUB, ROWS_PER_SUB)],
                        out_hbm.at[c, pl.ds(s * ROWS_PER_SUB, ROWS_PER_SUB)])

    return sc_layer


def _tc_pre(x, W1, degp):
    def body(x_ref, w_ref, degp_ref, hs_ref, dinv_ref):
        deg = degp_ref[0, :N, 0:1] + degp_ref[1, :N, 0:1] + 1.0
        dinv = lax.rsqrt(deg)
        h = jnp.dot(x_ref[...], w_ref[...],
                    preferred_element_type=jnp.float32)
        hs_ref[...] = h * dinv
        dinv_ref[...] = dinv

    return pl.pallas_call(
        body,
        out_shape=[jax.ShapeDtypeStruct((N, W1.shape[1]), jnp.float32),
                   jax.ShapeDtypeStruct((N, 1), jnp.float32)],
    )(x, W1, degp)


def _tc_mid(p, hs, dinv, b, g, be, Wn):
    def body(p_ref, hs_ref, dinv_ref, b_ref, g_ref, be_ref, w_ref, o_ref):
        agg = p_ref[0, :N] + p_ref[1, :N] + hs_ref[...]
        o = agg * dinv_ref[...] + b_ref[...]
        mean = jnp.mean(o, axis=0, keepdims=True)
        var = jnp.mean((o - mean) ** 2, axis=0, keepdims=True)
        o = (o - mean) / jnp.sqrt(var + EPS) * g_ref[...] + be_ref[...]
        o = jnp.maximum(o, 0.0)
        o_ref[...] = jnp.dot(o, w_ref[...],
                             preferred_element_type=jnp.float32) * dinv_ref[...]

    return pl.pallas_call(
        body,
        out_shape=jax.ShapeDtypeStruct((N, Wn.shape[1]), jnp.float32),
    )(p, hs, dinv, b, g, be, Wn)


def _tc_post(p, hs, dinv, b3, Wh1, bh1, Wh2, bh2):
    def body(p_ref, hs_ref, dinv_ref, b3_ref, wh1_ref, bh1_ref, wh2_ref,
             bh2_ref, emb_ref, log_ref):
        emb = ((p_ref[0, :N] + p_ref[1, :N] + hs_ref[...]) * dinv_ref[...]
               + b3_ref[...])
        f = jnp.maximum(jnp.dot(emb, wh1_ref[...],
                                preferred_element_type=jnp.float32)
                        + bh1_ref[...], 0.0)
        log_ref[...] = jnp.dot(f, wh2_ref[...],
                               preferred_element_type=jnp.float32) + bh2_ref[...]
        emb_ref[...] = emb

    return pl.pallas_call(
        body,
        out_shape=[jax.ShapeDtypeStruct((N, Wh1.shape[0]), jnp.float32),
                   jax.ShapeDtypeStruct((N, 1), jnp.float32)],
    )(p, hs, dinv, b3, Wh1, bh1, Wh2, bh2)


def kernel(x, edge_index, edge_weight, W1, b1, W2, b2, W3, b3,
           g1, be1, g2, be2, Wh1, bh1, Wh2, bh2):
    src = edge_index[0].astype(jnp.int32)
    dst = edge_index[1].astype(jnp.int32)
    pad = E_PAD - E
    src_p = jnp.concatenate(
        [src, jnp.zeros((pad,), jnp.int32)]).reshape(NW * K, B)
    dst_p = jnp.concatenate(
        [dst, jnp.zeros((pad,), jnp.int32)]).reshape(NW * K, B)
    ew_p = jnp.concatenate(
        [edge_weight.astype(jnp.float32),
         jnp.zeros((pad,), jnp.float32)]).reshape(NW * K, B)

    z16 = jnp.zeros((N_PAD, 16), jnp.float32)
    z64 = jnp.zeros((N_PAD, 64), jnp.float32)
    z32 = jnp.zeros((N_PAD, 32), jnp.float32)

    degp = _make_sc_deg()(dst_p, ew_p, z16)
    hs1, dinv = _tc_pre(x, W1, degp)

    p1 = _make_sc_layer(64)(src_p, dst_p, ew_p, hs1, z64)
    hs2 = _tc_mid(p1, hs1, dinv, b1.reshape(1, -1), g1.reshape(1, -1),
                  be1.reshape(1, -1), W2)
    p2 = _make_sc_layer(64)(src_p, dst_p, ew_p, hs2, z64)
    hs3 = _tc_mid(p2, hs2, dinv, b2.reshape(1, -1), g2.reshape(1, -1),
                  be2.reshape(1, -1), W3)
    p3 = _make_sc_layer(32)(src_p, dst_p, ew_p, hs3, z32)
    emb, logits = _tc_post(p3, hs3, dinv, b3.reshape(1, -1), Wh1,
                           bh1.reshape(1, -1), Wh2, bh2.reshape(1, -1))
    return emb, logits
